# baseline (speedup 1.0000x reference)
import functools

import jax
import jax.numpy as jnp
from jax import lax
from jax.experimental import pallas as pl
from jax.experimental.pallas import tpu as pltpu
from jax.experimental.pallas import tpu_sc as plsc

BATCH = 16384
DIM = 64
GRP = 8
GDIM = GRP * DIM
NUM_CORES = 2
NUM_SUBCORES = 16
NW = NUM_CORES * NUM_SUBCORES
BPW = BATCH // NW
LANES = 16
CHUNK = 64
NCH = BPW // CHUNK
NV = CHUNK // LANES
CPR = DIM // LANES
OROWS = BPW // GRP


def _gmf_body(uid_hbm, iid_hbm, utab_hbm, itab_hbm, out_hbm,
              uidx_v, iidx_v, gidx_v, tilebuf_v, out2_v, sem):
    c = lax.axis_index("c")
    s = lax.axis_index("s")
    wid = s * NUM_CORES + c
    base = wid * BPW

    pltpu.sync_copy(uid_hbm.at[pl.ds(base, BPW)], uidx_v)
    pltpu.sync_copy(iid_hbm.at[pl.ds(base, BPW)], iidx_v)

    def run_pass(tab_hbm, idx_v, combine):
        def gsplit(k, carry):
            sl = pl.ds(k * LANES, LANES)
            gidx_v[sl] = lax.shift_right_logical(idx_v[sl], 3)
            return carry

        lax.fori_loop(0, BPW // LANES, gsplit, 0)

        def chunk(k, carry):
            pltpu.async_copy(
                tab_hbm.at[gidx_v.at[pl.ds(k * CHUNK, CHUNK)]],
                tilebuf_v, sem).wait()

            def ext16(m, carry2):
                rvec = lax.bitwise_and(idx_v[pl.ds(k * CHUNK + m * LANES,
                                                   LANES)], 7)
                orow = k * (CHUNK // GRP) + m * (LANES // GRP)
                for l in range(LANES):
                    r = rvec[l]
                    for ci in range(CPR):
                        src = pl.ds(r * DIM + ci * LANES, LANES)
                        dst = pl.ds((l & 7) * DIM + ci * LANES, LANES)
                        combine(orow + (l >> 3), dst,
                                tilebuf_v[m * LANES + l, src])
                return carry2

            lax.fori_loop(0, NV, ext16, 0)
            return carry

        lax.fori_loop(0, NCH, chunk, 0)

    def put_user(row, dst, val):
        out2_v[row, dst] = val

    run_pass(utab_hbm, uidx_v, put_user)

    def mul_item(row, dst, val):
        out2_v[row, dst] = out2_v[row, dst] * val

    run_pass(itab_hbm, iidx_v, mul_item)

    pltpu.sync_copy(out2_v, out_hbm.at[pl.ds(wid * OROWS, OROWS)])


@jax.jit
def _gmf(user_id, item_id, mf_user_emb, mf_item_emb):
    utab = mf_user_emb.reshape(-1, GDIM)
    itab = mf_item_emb.reshape(-1, GDIM)
    mesh = plsc.VectorSubcoreMesh(core_axis_name="c", subcore_axis_name="s")
    f = pl.kernel(
        _gmf_body,
        mesh=mesh,
        compiler_params=pltpu.CompilerParams(use_tc_tiling_on_sc=True),
        out_type=jax.ShapeDtypeStruct((BATCH // GRP, GDIM), jnp.float32),
        scratch_types=[
            pltpu.VMEM((BPW,), jnp.int32),
            pltpu.VMEM((BPW,), jnp.int32),
            pltpu.VMEM((BPW,), jnp.int32),
            pltpu.VMEM((CHUNK, GDIM), jnp.float32),
            pltpu.VMEM((OROWS, GDIM), jnp.float32),
            pltpu.SemaphoreType.DMA,
        ],
    )
    return f(user_id, item_id, utab, itab).reshape(BATCH, DIM)


def kernel(user_id, item_id, mf_user_emb, mf_item_emb):
    return _gmf(user_id.astype(jnp.int32), item_id.astype(jnp.int32),
                mf_user_emb, mf_item_emb)


# speedup vs baseline: 2.2319x; 2.2319x over previous
import functools

import jax
import jax.numpy as jnp
from jax import lax
from jax.experimental import pallas as pl
from jax.experimental.pallas import tpu as pltpu
from jax.experimental.pallas import tpu_sc as plsc

BATCH = 16384
DIM = 64
GRP = 8
NUM_CORES = 2
NUM_SUBCORES = 16
NW = NUM_CORES * NUM_SUBCORES
BPW = BATCH // NW
LANES = 16
CHUNK = 16
NCH = BPW // CHUNK
CPR = DIM // LANES


def _gmf_body(uid_hbm, iid_hbm, utab_hbm, itab_hbm, out_hbm,
              uidx_v, iidx_v, ubuf_v, ibuf_v, out2_v, sem):
    c = lax.axis_index("c")
    s = lax.axis_index("s")
    wid = s * NUM_CORES + c
    base = wid * BPW

    pltpu.sync_copy(uid_hbm.at[pl.ds(base, BPW)], uidx_v)
    pltpu.sync_copy(iid_hbm.at[pl.ds(base, BPW)], iidx_v)

    def chunk(k, carry):
        sl16 = pl.ds(k * CHUNK, CHUNK)
        uvec = uidx_v[sl16]
        ivec = iidx_v[sl16]
        ug = lax.shift_right_logical(uvec, 3)
        ig = lax.shift_right_logical(ivec, 3)
        for l in range(CHUNK):
            pltpu.async_copy(utab_hbm.at[ug[l]], ubuf_v.at[l], sem)
            pltpu.async_copy(itab_hbm.at[ig[l]], ibuf_v.at[l], sem)
        pltpu.make_async_copy(utab_hbm.at[pl.ds(0, CHUNK)], ubuf_v,
                              sem).wait()
        pltpu.make_async_copy(utab_hbm.at[pl.ds(0, CHUNK)], ibuf_v,
                              sem).wait()

        ur = lax.bitwise_and(uvec, 7)
        ir = lax.bitwise_and(ivec, 7)
        for l in range(CHUNK):
            ru = ur[l]
            ri = ir[l]
            row = k * CHUNK + l
            for ci in range(CPR):
                sl = pl.ds(ci * LANES, LANES)
                out2_v[row, sl] = ubuf_v[l, ru, sl] * ibuf_v[l, ri, sl]
        return carry

    lax.fori_loop(0, NCH, chunk, 0)

    pltpu.sync_copy(out2_v, out_hbm.at[pl.ds(base, BPW)])


@jax.jit
def _gmf(user_id, item_id, mf_user_emb, mf_item_emb):
    utab3 = mf_user_emb.reshape(-1, GRP, DIM)
    itab3 = mf_item_emb.reshape(-1, GRP, DIM)
    mesh = plsc.VectorSubcoreMesh(core_axis_name="c", subcore_axis_name="s")
    f = pl.kernel(
        _gmf_body,
        mesh=mesh,
        compiler_params=pltpu.CompilerParams(use_tc_tiling_on_sc=True),
        out_type=jax.ShapeDtypeStruct((BATCH, DIM), jnp.float32),
        scratch_types=[
            pltpu.VMEM((BPW,), jnp.int32),
            pltpu.VMEM((BPW,), jnp.int32),
            pltpu.VMEM((CHUNK, GRP, DIM), jnp.float32),
            pltpu.VMEM((CHUNK, GRP, DIM), jnp.float32),
            pltpu.VMEM((BPW, DIM), jnp.float32),
            pltpu.SemaphoreType.DMA,
        ],
    )
    return f(user_id, item_id, utab3, itab3)


def kernel(user_id, item_id, mf_user_emb, mf_item_emb):
    return _gmf(user_id.astype(jnp.int32), item_id.astype(jnp.int32),
                mf_user_emb, mf_item_emb)


# speedup vs baseline: 2.2926x; 1.0272x over previous
import functools

import jax
import jax.numpy as jnp
from jax import lax
from jax.experimental import pallas as pl
from jax.experimental.pallas import tpu as pltpu
from jax.experimental.pallas import tpu_sc as plsc

BATCH = 16384
DIM = 64
GRP = 8
NUM_CORES = 2
NUM_SUBCORES = 16
NW = NUM_CORES * NUM_SUBCORES
BPW = BATCH // NW
LANES = 16
CHUNK = 16
HALF = BPW // 2
NCH_HALF = HALF // CHUNK
CPR = DIM // LANES


def _gmf_body(uid_hbm, iid_hbm, utab_hbm, itab_hbm, out_hbm,
              uidx_v, iidx_v, ubufa_v, ibufa_v, ubufb_v, ibufb_v, out2_v,
              sema, semb):
    c = lax.axis_index("c")
    s = lax.axis_index("s")
    wid = s * NUM_CORES + c
    base = wid * BPW

    pltpu.sync_copy(uid_hbm.at[pl.ds(base, BPW)], uidx_v)
    pltpu.sync_copy(iid_hbm.at[pl.ds(base, BPW)], iidx_v)

    def fire(ck, ubuf, ibuf, sem):
        sl16 = pl.ds(ck * CHUNK, CHUNK)
        ug = lax.shift_right_logical(uidx_v[sl16], 3)
        ig = lax.shift_right_logical(iidx_v[sl16], 3)
        for l in range(CHUNK):
            pltpu.async_copy(utab_hbm.at[ug[l]], ubuf.at[l], sem)
            pltpu.async_copy(itab_hbm.at[ig[l]], ibuf.at[l], sem)

    def drain(ubuf, sem):
        pltpu.make_async_copy(utab_hbm.at[pl.ds(0, CHUNK)], ubuf,
                              sem).wait()
        pltpu.make_async_copy(utab_hbm.at[pl.ds(0, CHUNK)], ubuf,
                              sem).wait()

    def extract(ck, h, ubuf, ibuf):
        sl16 = pl.ds(ck * CHUNK, CHUNK)
        ur = lax.bitwise_and(uidx_v[sl16], 7)
        ir = lax.bitwise_and(iidx_v[sl16], 7)
        lrow = (ck - h * NCH_HALF) * CHUNK
        for l in range(CHUNK):
            ru = ur[l]
            ri = ir[l]
            for ci in range(CPR):
                sl = pl.ds(ci * LANES, LANES)
                out2_v[lrow + l, sl] = ubuf[l, ru, sl] * ibuf[l, ri, sl]

    for h in range(2):
        first = h * NCH_HALF
        fire(first, ubufa_v, ibufa_v, sema)

        def pair(k4, carry):
            c0 = first + 2 * k4
            fire(c0 + 1, ubufb_v, ibufb_v, semb)
            drain(ubufa_v, sema)
            extract(c0, h, ubufa_v, ibufa_v)

            @pl.when(2 * k4 + 2 < NCH_HALF)
            def _():
                fire(c0 + 2, ubufa_v, ibufa_v, sema)

            drain(ubufb_v, semb)
            extract(c0 + 1, h, ubufb_v, ibufb_v)
            return carry

        lax.fori_loop(0, NCH_HALF // 2, pair, 0)
        pltpu.sync_copy(out2_v, out_hbm.at[pl.ds(base + h * HALF, HALF)])


@jax.jit
def _gmf(user_id, item_id, mf_user_emb, mf_item_emb):
    utab3 = mf_user_emb.reshape(-1, GRP, DIM)
    itab3 = mf_item_emb.reshape(-1, GRP, DIM)
    mesh = plsc.VectorSubcoreMesh(core_axis_name="c", subcore_axis_name="s")
    f = pl.kernel(
        _gmf_body,
        mesh=mesh,
        compiler_params=pltpu.CompilerParams(use_tc_tiling_on_sc=True),
        out_type=jax.ShapeDtypeStruct((BATCH, DIM), jnp.float32),
        scratch_types=[
            pltpu.VMEM((BPW,), jnp.int32),
            pltpu.VMEM((BPW,), jnp.int32),
            pltpu.VMEM((CHUNK, GRP, DIM), jnp.float32),
            pltpu.VMEM((CHUNK, GRP, DIM), jnp.float32),
            pltpu.VMEM((CHUNK, GRP, DIM), jnp.float32),
            pltpu.VMEM((CHUNK, GRP, DIM), jnp.float32),
            pltpu.VMEM((HALF, DIM), jnp.float32),
            pltpu.SemaphoreType.DMA,
            pltpu.SemaphoreType.DMA,
        ],
    )
    return f(user_id, item_id, utab3, itab3)


def kernel(user_id, item_id, mf_user_emb, mf_item_emb):
    return _gmf(user_id.astype(jnp.int32), item_id.astype(jnp.int32),
                mf_user_emb, mf_item_emb)


# speedup vs baseline: 2.2944x; 1.0008x over previous
import jax
import jax.numpy as jnp
from jax import lax
from jax.experimental import pallas as pl
from jax.experimental.pallas import tpu as pltpu
from jax.experimental.pallas import tpu_sc as plsc

BATCH = 16384
DIM = 64
GRP = 8
NUM_CORES = 2
NUM_SUBCORES = 16
NW = NUM_CORES * NUM_SUBCORES
BPW = BATCH // NW
LANES = 16
CHUNK = 16
HALF = BPW // 2
NCH_HALF = HALF // CHUNK
CPR = DIM // LANES


def _gmf_body(uid_hbm, iid_hbm, utab_hbm, itab_hbm, out_hbm,
              uidx_v, iidx_v, ubufa_v, ibufa_v, ubufb_v, ibufb_v, out2_v,
              sema, semb):
    c = lax.axis_index("c")
    s = lax.axis_index("s")
    wid = s * NUM_CORES + c
    base = wid * BPW

    pltpu.sync_copy(uid_hbm.at[pl.ds(base, BPW)], uidx_v)
    pltpu.sync_copy(iid_hbm.at[pl.ds(base, BPW)], iidx_v)

    def fire(ck, ubuf, ibuf, sem):
        sl16 = pl.ds(ck * CHUNK, CHUNK)
        ug = lax.shift_right_logical(uidx_v[sl16], 3)
        ig = lax.shift_right_logical(iidx_v[sl16], 3)
        for l in range(CHUNK):
            pltpu.async_copy(utab_hbm.at[ug[l]], ubuf.at[l], sem)
            pltpu.async_copy(itab_hbm.at[ig[l]], ibuf.at[l], sem)

    def drain(ubuf, sem):
        pltpu.make_async_copy(utab_hbm.at[pl.ds(0, CHUNK)], ubuf,
                              sem).wait()
        pltpu.make_async_copy(utab_hbm.at[pl.ds(0, CHUNK)], ubuf,
                              sem).wait()

    def extract(ck, h, ubuf, ibuf):
        sl16 = pl.ds(ck * CHUNK, CHUNK)
        ur = lax.bitwise_and(uidx_v[sl16], 7)
        ir = lax.bitwise_and(iidx_v[sl16], 7)
        lrow = (ck - h * NCH_HALF) * CHUNK
        for l in range(CHUNK):
            ru = ur[l]
            ri = ir[l]
            for ci in range(CPR):
                sl = pl.ds(ci * LANES, LANES)
                out2_v[lrow + l, sl] = ubuf[l, ru, sl] * ibuf[l, ri, sl]

    for h in range(2):
        first = h * NCH_HALF
        fire(first, ubufa_v, ibufa_v, sema)

        def pair(k4, carry):
            c0 = first + 2 * k4
            fire(c0 + 1, ubufb_v, ibufb_v, semb)
            drain(ubufa_v, sema)
            extract(c0, h, ubufa_v, ibufa_v)

            @pl.when(2 * k4 + 2 < NCH_HALF)
            def _():
                fire(c0 + 2, ubufa_v, ibufa_v, sema)

            drain(ubufb_v, semb)
            extract(c0 + 1, h, ubufb_v, ibufb_v)
            return carry

        lax.fori_loop(0, NCH_HALF // 2, pair, 0)
        pltpu.sync_copy(out2_v, out_hbm.at[pl.ds(base + h * HALF, HALF)])


@jax.jit
def _gmf(user_id, item_id, mf_user_emb, mf_item_emb):
    utab3 = mf_user_emb.reshape(-1, GRP, DIM)
    itab3 = mf_item_emb.reshape(-1, GRP, DIM)
    mesh = plsc.VectorSubcoreMesh(core_axis_name="c", subcore_axis_name="s")
    f = pl.kernel(
        _gmf_body,
        mesh=mesh,
        compiler_params=pltpu.CompilerParams(use_tc_tiling_on_sc=True),
        out_type=jax.ShapeDtypeStruct((BATCH, DIM), jnp.float32),
        scratch_types=[
            pltpu.VMEM((BPW,), jnp.int32),
            pltpu.VMEM((BPW,), jnp.int32),
            pltpu.VMEM((CHUNK, GRP, DIM), jnp.float32),
            pltpu.VMEM((CHUNK, GRP, DIM), jnp.float32),
            pltpu.VMEM((CHUNK, GRP, DIM), jnp.float32),
            pltpu.VMEM((CHUNK, GRP, DIM), jnp.float32),
            pltpu.VMEM((HALF, DIM), jnp.float32),
            pltpu.SemaphoreType.DMA,
            pltpu.SemaphoreType.DMA,
        ],
    )
    return f(user_id, item_id, utab3, itab3)


def kernel(user_id, item_id, mf_user_emb, mf_item_emb):
    return _gmf(user_id.astype(jnp.int32), item_id.astype(jnp.int32),
                mf_user_emb, mf_item_emb)
